# baseline (speedup 1.0000x reference)
import functools

import jax
import jax.numpy as jnp
from jax.experimental import pallas as pl
from jax.experimental.pallas import tpu as pltpu

VIEWS = 32
K = 5
IMG = 3 * 34 * 18
FEAT = 512

GB = 8
ROWS = GB * VIEWS


def _prep_body(v_ref, cw_ref, cb_ref, w1_ref, b1_ref, w2_ref, b2_ref,
               w3_ref, b3_ref, linT_ref, lb_ref, a_out, bias_out, wcl_out):
    V = v_ref[...]
    G = jax.lax.dot_general(V, V, (((1,), (1,)), ((), ())),
                            preferred_element_type=jnp.float32)
    ii = jax.lax.broadcasted_iota(jnp.int32, (VIEWS, VIEWS), 0)
    jj = jax.lax.broadcasted_iota(jnp.int32, (VIEWS, VIEWS), 1)
    diagmask = ii == jj
    xx_col = jnp.sum(jnp.where(diagmask, G, 0.0), axis=1, keepdims=True)
    xx_row = jnp.sum(jnp.where(diagmask, G, 0.0), axis=0, keepdims=True)
    nd = 2.0 * G - xx_col - xx_row

    b1 = b1_ref[...]
    b2 = b2_ref[...]
    b3 = b3_ref[...]
    A = jnp.zeros((VIEWS, VIEWS), jnp.float32)
    v0 = None
    for k in range(K):
        m = jnp.max(nd, axis=1, keepdims=True)
        cand = jnp.where(nd >= m, jj, jnp.int32(2 ** 30))
        idxk = jnp.min(cand, axis=1, keepdims=True)
        onehot = jnp.where(jj == idxk, 1.0, 0.0)
        vk = jnp.dot(onehot, V, preferred_element_type=jnp.float32)
        if k == 0:
            v0 = vk
        diff = v0 - vk
        nrm = jnp.sqrt(jnp.sum(diff * diff, axis=1, keepdims=True))
        h = (jnp.dot(v0, w1_ref[0:3, :], preferred_element_type=jnp.float32)
             + jnp.dot(vk, w1_ref[3:6, :], preferred_element_type=jnp.float32)
             + jnp.dot(diff, w1_ref[6:9, :], preferred_element_type=jnp.float32)
             + nrm * w1_ref[9:10, :] + b1)
        h = jnp.maximum(h, 0.0)
        h = jnp.maximum(jnp.dot(h, w2_ref[...],
                                preferred_element_type=jnp.float32) + b2, 0.0)
        wk = jnp.dot(h, w3_ref[...], preferred_element_type=jnp.float32) + b3
        A = A + wk * onehot
        nd = jnp.where(jj == idxk, jnp.float32(-1e30), nd)
    a_out[...] = A

    s = jnp.sum(A, axis=1, keepdims=True)
    t = jnp.sum(linT_ref[...], axis=0, keepdims=True)
    bias_out[...] = cb_ref[...] * s * t + lb_ref[...]

    L1 = jax.lax.broadcasted_iota(jnp.int32, (IMG, 1), 0)
    mcol = jax.lax.broadcasted_iota(jnp.int32, (1, FEAT), 1)
    r_ = mcol // 16
    c_ = mcol % 16
    Wc = jnp.zeros((IMG, FEAT), jnp.float32)
    for ci in range(3):
        for dr in range(3):
            for dc in range(3):
                tgt = ci * 612 + (r_ + dr) * 18 + (c_ + dc)
                kk = ci * 9 + dr * 3 + dc
                Wc = Wc + jnp.where(L1 == tgt, cw_ref[0:1, kk:kk + 1], 0.0)
    wcl_out[...] = jnp.dot(Wc, linT_ref[...], preferred_element_type=jnp.float32)


AROWS = 2048


def _mega_body(nsteps, n_total, x_ref, wcl_ref, a_ref, bias_ref, g_ref, b_ref,
               o_ref, f_scr, st_scr):
    i = pl.program_id(0)

    @pl.when(i < nsteps)
    def _main():
        P = jnp.dot(x_ref[...], wcl_ref[...],
                    preferred_element_type=jnp.float32)
        Av = a_ref[...]
        Bv = bias_ref[...]
        ssum = jnp.zeros((1, FEAT), jnp.float32)
        ssq = jnp.zeros((1, FEAT), jnp.float32)
        base = i * ROWS
        for g in range(GB):
            Z = jnp.dot(Av, P[g * VIEWS:(g + 1) * VIEWS, :],
                        preferred_element_type=jnp.float32) + Bv
            Fg = jnp.maximum(Z, 0.0)
            f_scr[pl.ds(base + g * VIEWS, VIEWS), :] = Fg
            ssum = ssum + jnp.sum(Fg, axis=0, keepdims=True)
            ssq = ssq + jnp.sum(Fg * Fg, axis=0, keepdims=True)

        @pl.when(i == 0)
        def _init():
            st_scr[0:1, :] = ssum
            st_scr[1:2, :] = ssq

        @pl.when(i > 0)
        def _acc():
            st_scr[0:1, :] = st_scr[0:1, :] + ssum
            st_scr[1:2, :] = st_scr[1:2, :] + ssq

    @pl.when(i >= nsteps)
    def _apply():
        j = i - nsteps
        inv_n = jnp.float32(1.0 / n_total)
        mean = st_scr[0:1, :] * inv_n
        msq = st_scr[1:2, :] * inv_n
        var = msq - mean * mean
        scale = g_ref[...] * jax.lax.rsqrt(var + 1e-5)
        shift = b_ref[...] - mean * scale
        o_ref[...] = f_scr[pl.ds(j * AROWS, AROWS), :] * scale + shift


def kernel(x, vertices, conv_w, conv_b, r_w1, r_b1, r_w2, r_b2, r_w3, r_b3,
           lin_w, lin_b, bn_gamma, bn_beta):
    n = x.shape[0]
    xf = x.reshape(n, IMG)
    lin_wT = lin_w.T

    a_mat, bias, wcl = pl.pallas_call(
        _prep_body,
        out_shape=[
            jax.ShapeDtypeStruct((VIEWS, VIEWS), jnp.float32),
            jax.ShapeDtypeStruct((VIEWS, FEAT), jnp.float32),
            jax.ShapeDtypeStruct((IMG, FEAT), jnp.float32),
        ],
    )(vertices, conv_w.reshape(1, 27), conv_b.reshape(1, 1),
      r_w1.T, r_b1.reshape(1, 10), r_w2.T, r_b2.reshape(1, 10),
      r_w3.T, r_b3.reshape(1, 1), lin_wT, lin_b.reshape(1, FEAT))

    nsteps = n // ROWS
    napply = n // AROWS
    out = pl.pallas_call(
        functools.partial(_mega_body, nsteps, n),
        grid=(nsteps + napply,),
        in_specs=[
            pl.BlockSpec((ROWS, IMG), lambda i: (jnp.minimum(i, nsteps - 1), 0)),
            pl.BlockSpec((IMG, FEAT), lambda i: (0, 0)),
            pl.BlockSpec((VIEWS, VIEWS), lambda i: (0, 0)),
            pl.BlockSpec((VIEWS, FEAT), lambda i: (0, 0)),
            pl.BlockSpec((1, FEAT), lambda i: (0, 0)),
            pl.BlockSpec((1, FEAT), lambda i: (0, 0)),
        ],
        out_specs=pl.BlockSpec(
            (AROWS, FEAT), lambda i: (jnp.maximum(i - nsteps, 0), 0)),
        out_shape=jax.ShapeDtypeStruct((n, FEAT), jnp.float32),
        scratch_shapes=[
            pltpu.VMEM((n, FEAT), jnp.float32),
            pltpu.VMEM((2, FEAT), jnp.float32),
        ],
        compiler_params=pltpu.CompilerParams(
            dimension_semantics=("arbitrary",)),
    )(xf, wcl, a_mat, bias,
      bn_gamma.reshape(1, FEAT), bn_beta.reshape(1, FEAT))

    return out.reshape(n, 1, FEAT)


# speedup vs baseline: 1.2458x; 1.2458x over previous
import functools

import jax
import jax.numpy as jnp
from jax.experimental import pallas as pl
from jax.experimental.pallas import tpu as pltpu

VIEWS = 32
K = 5
IMG = 3 * 34 * 18
FEAT = 512

GB = 8
ROWS = GB * VIEWS


def _prep_body(v_ref, cw_ref, cb_ref, w1_ref, b1_ref, w2_ref, b2_ref,
               w3_ref, b3_ref, linT_ref, lb_ref, a_out, bias_out, wcl_out):
    V = v_ref[...]
    G = jax.lax.dot_general(V, V, (((1,), (1,)), ((), ())),
                            preferred_element_type=jnp.float32)
    ii = jax.lax.broadcasted_iota(jnp.int32, (VIEWS, VIEWS), 0)
    jj = jax.lax.broadcasted_iota(jnp.int32, (VIEWS, VIEWS), 1)
    diagmask = ii == jj
    xx_col = jnp.sum(jnp.where(diagmask, G, 0.0), axis=1, keepdims=True)
    xx_row = jnp.sum(jnp.where(diagmask, G, 0.0), axis=0, keepdims=True)
    nd = 2.0 * G - xx_col - xx_row

    b1 = b1_ref[...]
    b2 = b2_ref[...]
    b3 = b3_ref[...]
    A = jnp.zeros((VIEWS, VIEWS), jnp.float32)
    v0 = None
    for k in range(K):
        m = jnp.max(nd, axis=1, keepdims=True)
        cand = jnp.where(nd >= m, jj, jnp.int32(2 ** 30))
        idxk = jnp.min(cand, axis=1, keepdims=True)
        onehot = jnp.where(jj == idxk, 1.0, 0.0)
        vk = jnp.dot(onehot, V, preferred_element_type=jnp.float32)
        if k == 0:
            v0 = vk
        diff = v0 - vk
        nrm = jnp.sqrt(jnp.sum(diff * diff, axis=1, keepdims=True))
        h = (jnp.dot(v0, w1_ref[0:3, :], preferred_element_type=jnp.float32)
             + jnp.dot(vk, w1_ref[3:6, :], preferred_element_type=jnp.float32)
             + jnp.dot(diff, w1_ref[6:9, :], preferred_element_type=jnp.float32)
             + nrm * w1_ref[9:10, :] + b1)
        h = jnp.maximum(h, 0.0)
        h = jnp.maximum(jnp.dot(h, w2_ref[...],
                                preferred_element_type=jnp.float32) + b2, 0.0)
        wk = jnp.dot(h, w3_ref[...], preferred_element_type=jnp.float32) + b3
        A = A + wk * onehot
        nd = jnp.where(jj == idxk, jnp.float32(-1e30), nd)
    a_out[...] = A

    s = jnp.sum(A, axis=1, keepdims=True)
    t = jnp.sum(linT_ref[...], axis=0, keepdims=True)
    bias_out[...] = cb_ref[...] * s * t + lb_ref[...]

    L1 = jax.lax.broadcasted_iota(jnp.int32, (IMG, 1), 0)
    mcol = jax.lax.broadcasted_iota(jnp.int32, (1, FEAT), 1)
    r_ = mcol // 16
    c_ = mcol % 16
    Wc = jnp.zeros((IMG, FEAT), jnp.float32)
    for ci in range(3):
        for dr in range(3):
            for dc in range(3):
                tgt = ci * 612 + (r_ + dr) * 18 + (c_ + dc)
                kk = ci * 9 + dr * 3 + dc
                Wc = Wc + jnp.where(L1 == tgt, cw_ref[0:1, kk:kk + 1], 0.0)
    wcl_out[...] = jnp.dot(Wc, linT_ref[...], preferred_element_type=jnp.float32)


AROWS = 2048


def _mega_body(nsteps, n_total, x_ref, wcl_ref, a_ref, bias_ref, g_ref, b_ref,
               o_ref, f_scr, st_scr):
    i = pl.program_id(0)

    @pl.when(i < nsteps)
    def _main():
        P = jnp.dot(x_ref[...], wcl_ref[...],
                    preferred_element_type=jnp.float32)
        Av = a_ref[...]
        Bv = bias_ref[...]
        ssum = jnp.zeros((1, FEAT), jnp.float32)
        ssq = jnp.zeros((1, FEAT), jnp.float32)
        base = i * ROWS
        for g in range(GB):
            Z = jnp.dot(Av, P[g * VIEWS:(g + 1) * VIEWS, :],
                        preferred_element_type=jnp.float32) + Bv
            Fg = jnp.maximum(Z, 0.0)
            f_scr[pl.ds(base + g * VIEWS, VIEWS), :] = Fg
            ssum = ssum + jnp.sum(Fg, axis=0, keepdims=True)
            ssq = ssq + jnp.sum(Fg * Fg, axis=0, keepdims=True)

        @pl.when(i == 0)
        def _init():
            st_scr[0:1, :] = ssum
            st_scr[1:2, :] = ssq

        @pl.when(i > 0)
        def _acc():
            st_scr[0:1, :] = st_scr[0:1, :] + ssum
            st_scr[1:2, :] = st_scr[1:2, :] + ssq

    @pl.when(i >= nsteps)
    def _apply():
        j = i - nsteps
        inv_n = jnp.float32(1.0 / n_total)
        mean = st_scr[0:1, :] * inv_n
        msq = st_scr[1:2, :] * inv_n
        var = msq - mean * mean
        scale = g_ref[...] * jax.lax.rsqrt(var + 1e-5)
        shift = b_ref[...] - mean * scale
        o_ref[...] = f_scr[pl.ds(j * AROWS, AROWS), :] * scale + shift


def kernel(x, vertices, conv_w, conv_b, r_w1, r_b1, r_w2, r_b2, r_w3, r_b3,
           lin_w, lin_b, bn_gamma, bn_beta):
    n = x.shape[0]
    xf = x.reshape(n, IMG).astype(jnp.bfloat16)
    lin_wT = lin_w.T

    a_mat, bias, wcl = pl.pallas_call(
        _prep_body,
        out_shape=[
            jax.ShapeDtypeStruct((VIEWS, VIEWS), jnp.float32),
            jax.ShapeDtypeStruct((VIEWS, FEAT), jnp.float32),
            jax.ShapeDtypeStruct((IMG, FEAT), jnp.float32),
        ],
    )(vertices, conv_w.reshape(1, 27), conv_b.reshape(1, 1),
      r_w1.T, r_b1.reshape(1, 10), r_w2.T, r_b2.reshape(1, 10),
      r_w3.T, r_b3.reshape(1, 1), lin_wT, lin_b.reshape(1, FEAT))

    wcl = wcl.astype(jnp.bfloat16)
    nsteps = n // ROWS
    napply = n // AROWS
    out = pl.pallas_call(
        functools.partial(_mega_body, nsteps, n),
        grid=(nsteps + napply,),
        in_specs=[
            pl.BlockSpec((ROWS, IMG), lambda i: (jnp.minimum(i, nsteps - 1), 0)),
            pl.BlockSpec((IMG, FEAT), lambda i: (0, 0)),
            pl.BlockSpec((VIEWS, VIEWS), lambda i: (0, 0)),
            pl.BlockSpec((VIEWS, FEAT), lambda i: (0, 0)),
            pl.BlockSpec((1, FEAT), lambda i: (0, 0)),
            pl.BlockSpec((1, FEAT), lambda i: (0, 0)),
        ],
        out_specs=pl.BlockSpec(
            (AROWS, FEAT), lambda i: (jnp.maximum(i - nsteps, 0), 0)),
        out_shape=jax.ShapeDtypeStruct((n, FEAT), jnp.float32),
        scratch_shapes=[
            pltpu.VMEM((n, FEAT), jnp.float32),
            pltpu.VMEM((2, FEAT), jnp.float32),
        ],
        compiler_params=pltpu.CompilerParams(
            dimension_semantics=("arbitrary",)),
    )(xf, wcl, a_mat, bias,
      bn_gamma.reshape(1, FEAT), bn_beta.reshape(1, FEAT))

    return out.reshape(n, 1, FEAT)


# speedup vs baseline: 1.2474x; 1.0013x over previous
import functools

import jax
import jax.numpy as jnp
from jax.experimental import pallas as pl
from jax.experimental.pallas import tpu as pltpu

VIEWS = 32
K = 5
IMG = 3 * 34 * 18
FEAT = 512

GB = 8
ROWS = GB * VIEWS


def _prep_body(v_ref, cw_ref, cb_ref, w1_ref, b1_ref, w2_ref, b2_ref,
               w3_ref, b3_ref, linT_ref, lb_ref, a_out, bias_out, wcl_out):
    V = v_ref[...]
    hi = jax.lax.Precision.HIGHEST
    Vb = V.astype(jnp.bfloat16)
    G = jax.lax.dot_general(Vb, Vb, (((1,), (1,)), ((), ())),
                            preferred_element_type=jnp.float32)
    ii = jax.lax.broadcasted_iota(jnp.int32, (VIEWS, VIEWS), 0)
    jj = jax.lax.broadcasted_iota(jnp.int32, (VIEWS, VIEWS), 1)
    xx = jnp.sum(V * V, axis=1, keepdims=True)
    diagmask = ii == jj
    xx_row = jnp.sum(jnp.where(diagmask, xx, 0.0), axis=0, keepdims=True)
    nd = 2.0 * G - xx - xx_row

    b1 = b1_ref[...]
    b2 = b2_ref[...]
    b3 = b3_ref[...]
    A = jnp.zeros((VIEWS, VIEWS), jnp.float32)
    v0 = None
    for k in range(K):
        m = jnp.max(nd, axis=1, keepdims=True)
        cand = jnp.where(nd >= m, jj, jnp.int32(2 ** 30))
        idxk = jnp.min(cand, axis=1, keepdims=True)
        onehot = jnp.where(jj == idxk, 1.0, 0.0)
        vk = jnp.dot(onehot, V, precision=hi,
                     preferred_element_type=jnp.float32)
        if k == 0:
            v0 = vk
        diff = v0 - vk
        nrm = jnp.sqrt(jnp.sum(diff * diff, axis=1, keepdims=True))
        h = (jnp.dot(v0, w1_ref[0:3, :], precision=hi,
                     preferred_element_type=jnp.float32)
             + jnp.dot(vk, w1_ref[3:6, :], precision=hi,
                       preferred_element_type=jnp.float32)
             + jnp.dot(diff, w1_ref[6:9, :], precision=hi,
                       preferred_element_type=jnp.float32)
             + nrm * w1_ref[9:10, :] + b1)
        h = jnp.maximum(h, 0.0)
        h = jnp.maximum(jnp.dot(h, w2_ref[...], precision=hi,
                                preferred_element_type=jnp.float32) + b2, 0.0)
        wk = jnp.dot(h, w3_ref[...], precision=hi,
                     preferred_element_type=jnp.float32) + b3
        A = A + wk * onehot
        nd = jnp.where(jj == idxk, jnp.float32(-1e30), nd)
    a_out[...] = A

    s = jnp.sum(A, axis=1, keepdims=True)
    t = jnp.sum(linT_ref[...], axis=0, keepdims=True)
    bias_out[...] = cb_ref[...] * s * t + lb_ref[...]

    L1 = jax.lax.broadcasted_iota(jnp.int32, (IMG, 1), 0)
    mcol = jax.lax.broadcasted_iota(jnp.int32, (1, FEAT), 1)
    r_ = mcol // 16
    c_ = mcol % 16
    Wc = jnp.zeros((IMG, FEAT), jnp.float32)
    for ci in range(3):
        for dr in range(3):
            for dc in range(3):
                tgt = ci * 612 + (r_ + dr) * 18 + (c_ + dc)
                kk = ci * 9 + dr * 3 + dc
                Wc = Wc + jnp.where(L1 == tgt, cw_ref[0:1, kk:kk + 1], 0.0)
    wcl_out[...] = jnp.dot(Wc, linT_ref[...], preferred_element_type=jnp.float32)


AROWS = 2048


def _mega_body(nsteps, n_total, x_ref, wcl_ref, a_ref, bias_ref, g_ref, b_ref,
               o_ref, f_scr, st_scr):
    i = pl.program_id(0)

    @pl.when(i < nsteps)
    def _main():
        P = jnp.dot(x_ref[...], wcl_ref[...],
                    preferred_element_type=jnp.float32)
        Av = a_ref[...]
        Bv = bias_ref[...]
        ssum = jnp.zeros((1, FEAT), jnp.float32)
        ssq = jnp.zeros((1, FEAT), jnp.float32)
        base = i * ROWS
        for g in range(GB):
            Z = jnp.dot(Av, P[g * VIEWS:(g + 1) * VIEWS, :],
                        preferred_element_type=jnp.float32) + Bv
            Fg = jnp.maximum(Z, 0.0)
            f_scr[pl.ds(base + g * VIEWS, VIEWS), :] = Fg
            ssum = ssum + jnp.sum(Fg, axis=0, keepdims=True)
            ssq = ssq + jnp.sum(Fg * Fg, axis=0, keepdims=True)

        @pl.when(i == 0)
        def _init():
            st_scr[0:1, :] = ssum
            st_scr[1:2, :] = ssq

        @pl.when(i > 0)
        def _acc():
            st_scr[0:1, :] = st_scr[0:1, :] + ssum
            st_scr[1:2, :] = st_scr[1:2, :] + ssq

    @pl.when(i >= nsteps)
    def _apply():
        j = i - nsteps
        inv_n = jnp.float32(1.0 / n_total)
        mean = st_scr[0:1, :] * inv_n
        msq = st_scr[1:2, :] * inv_n
        var = msq - mean * mean
        scale = g_ref[...] * jax.lax.rsqrt(var + 1e-5)
        shift = b_ref[...] - mean * scale
        o_ref[...] = f_scr[pl.ds(j * AROWS, AROWS), :] * scale + shift


def kernel(x, vertices, conv_w, conv_b, r_w1, r_b1, r_w2, r_b2, r_w3, r_b3,
           lin_w, lin_b, bn_gamma, bn_beta):
    n = x.shape[0]
    xf = x.reshape(n, IMG).astype(jnp.bfloat16)
    lin_wT = lin_w.T

    a_mat, bias, wcl = pl.pallas_call(
        _prep_body,
        out_shape=[
            jax.ShapeDtypeStruct((VIEWS, VIEWS), jnp.float32),
            jax.ShapeDtypeStruct((VIEWS, FEAT), jnp.float32),
            jax.ShapeDtypeStruct((IMG, FEAT), jnp.float32),
        ],
    )(vertices, conv_w.reshape(1, 27), conv_b.reshape(1, 1),
      r_w1.T, r_b1.reshape(1, 10), r_w2.T, r_b2.reshape(1, 10),
      r_w3.T, r_b3.reshape(1, 1), lin_wT, lin_b.reshape(1, FEAT))

    wcl = wcl.astype(jnp.bfloat16)
    nsteps = n // ROWS
    napply = n // AROWS
    out = pl.pallas_call(
        functools.partial(_mega_body, nsteps, n),
        grid=(nsteps + napply,),
        in_specs=[
            pl.BlockSpec((ROWS, IMG), lambda i: (jnp.minimum(i, nsteps - 1), 0)),
            pl.BlockSpec((IMG, FEAT), lambda i: (0, 0)),
            pl.BlockSpec((VIEWS, VIEWS), lambda i: (0, 0)),
            pl.BlockSpec((VIEWS, FEAT), lambda i: (0, 0)),
            pl.BlockSpec((1, FEAT), lambda i: (0, 0)),
            pl.BlockSpec((1, FEAT), lambda i: (0, 0)),
        ],
        out_specs=pl.BlockSpec(
            (AROWS, FEAT), lambda i: (jnp.maximum(i - nsteps, 0), 0)),
        out_shape=jax.ShapeDtypeStruct((n, FEAT), jnp.float32),
        scratch_shapes=[
            pltpu.VMEM((n, FEAT), jnp.float32),
            pltpu.VMEM((2, FEAT), jnp.float32),
        ],
        compiler_params=pltpu.CompilerParams(
            dimension_semantics=("arbitrary",)),
    )(xf, wcl, a_mat, bias,
      bn_gamma.reshape(1, FEAT), bn_beta.reshape(1, FEAT))

    return out.reshape(n, 1, FEAT)


# speedup vs baseline: 1.2477x; 1.0002x over previous
import functools

import jax
import jax.numpy as jnp
from jax.experimental import pallas as pl
from jax.experimental.pallas import tpu as pltpu

VIEWS = 32
K = 5
IMG = 3 * 34 * 18
FEAT = 512

GB = 8
ROWS = GB * VIEWS


def _prep_body(v_ref, cw_ref, cb_ref, w1_ref, b1_ref, w2_ref, b2_ref,
               w3_ref, b3_ref, lin_ref, lb_ref, a_out, bias_out, wcl_out):
    V = v_ref[...]
    hi = jax.lax.Precision.HIGHEST
    Vb = V.astype(jnp.bfloat16)
    G = jax.lax.dot_general(Vb, Vb, (((1,), (1,)), ((), ())),
                            preferred_element_type=jnp.float32)
    ii = jax.lax.broadcasted_iota(jnp.int32, (VIEWS, VIEWS), 0)
    jj = jax.lax.broadcasted_iota(jnp.int32, (VIEWS, VIEWS), 1)
    xx = jnp.sum(V * V, axis=1, keepdims=True)
    diagmask = ii == jj
    xx_row = jnp.sum(jnp.where(diagmask, xx, 0.0), axis=0, keepdims=True)
    nd = 2.0 * G - xx - xx_row

    b1 = b1_ref[...]
    b2 = b2_ref[...]
    b3 = b3_ref[...]
    A = jnp.zeros((VIEWS, VIEWS), jnp.float32)
    v0 = None
    for k in range(K):
        m = jnp.max(nd, axis=1, keepdims=True)
        cand = jnp.where(nd >= m, jj, jnp.int32(2 ** 30))
        idxk = jnp.min(cand, axis=1, keepdims=True)
        onehot = jnp.where(jj == idxk, 1.0, 0.0)
        vk = jnp.dot(onehot, V, precision=hi,
                     preferred_element_type=jnp.float32)
        if k == 0:
            v0 = vk
        diff = v0 - vk
        nrm = jnp.sqrt(jnp.sum(diff * diff, axis=1, keepdims=True))
        h = (jnp.dot(v0, w1_ref[0:3, :], precision=hi,
                     preferred_element_type=jnp.float32)
             + jnp.dot(vk, w1_ref[3:6, :], precision=hi,
                       preferred_element_type=jnp.float32)
             + jnp.dot(diff, w1_ref[6:9, :], precision=hi,
                       preferred_element_type=jnp.float32)
             + nrm * w1_ref[9:10, :] + b1)
        h = jnp.maximum(h, 0.0)
        h = jnp.maximum(jnp.dot(h, w2_ref[...], precision=hi,
                                preferred_element_type=jnp.float32) + b2, 0.0)
        wk = jnp.dot(h, w3_ref[...], precision=hi,
                     preferred_element_type=jnp.float32) + b3
        A = A + wk * onehot
        nd = jnp.where(jj == idxk, jnp.float32(-1e30), nd)
    a_out[...] = A

    s = jnp.sum(A, axis=1, keepdims=True)
    t = jax.lax.dot_general(jnp.ones((1, FEAT), jnp.float32), lin_ref[...],
                            (((1,), (1,)), ((), ())), precision=hi,
                            preferred_element_type=jnp.float32)
    bias_out[...] = cb_ref[...] * s * t + lb_ref[...]

    L1 = jax.lax.broadcasted_iota(jnp.int32, (IMG, 1), 0)
    mcol = jax.lax.broadcasted_iota(jnp.int32, (1, FEAT), 1)
    r_ = mcol // 16
    c_ = mcol % 16
    Wc = jnp.zeros((IMG, FEAT), jnp.float32)
    for ci in range(3):
        for dr in range(3):
            for dc in range(3):
                tgt = ci * 612 + (r_ + dr) * 18 + (c_ + dc)
                kk = ci * 9 + dr * 3 + dc
                Wc = Wc + jnp.where(L1 == tgt, cw_ref[0:1, kk:kk + 1], 0.0)
    wcl_out[...] = jax.lax.dot_general(Wc, lin_ref[...],
                                       (((1,), (1,)), ((), ())),
                                       preferred_element_type=jnp.float32)


AROWS = 2048


def _mega_body(nsteps, n_total, x_ref, wcl_ref, a_ref, bias_ref, g_ref, b_ref,
               o_ref, f_scr, st_scr):
    i = pl.program_id(0)

    @pl.when(i < nsteps)
    def _main():
        P = jnp.dot(x_ref[...], wcl_ref[...],
                    preferred_element_type=jnp.float32)
        Av = a_ref[...]
        Bv = bias_ref[...]
        ssum = jnp.zeros((1, FEAT), jnp.float32)
        ssq = jnp.zeros((1, FEAT), jnp.float32)
        base = i * ROWS
        for g in range(GB):
            Z = jnp.dot(Av, P[g * VIEWS:(g + 1) * VIEWS, :],
                        preferred_element_type=jnp.float32) + Bv
            Fg = jnp.maximum(Z, 0.0)
            f_scr[pl.ds(base + g * VIEWS, VIEWS), :] = Fg
            ssum = ssum + jnp.sum(Fg, axis=0, keepdims=True)
            ssq = ssq + jnp.sum(Fg * Fg, axis=0, keepdims=True)

        @pl.when(i == 0)
        def _init():
            st_scr[0:1, :] = ssum
            st_scr[1:2, :] = ssq

        @pl.when(i > 0)
        def _acc():
            st_scr[0:1, :] = st_scr[0:1, :] + ssum
            st_scr[1:2, :] = st_scr[1:2, :] + ssq

    @pl.when(i >= nsteps)
    def _apply():
        j = i - nsteps
        inv_n = jnp.float32(1.0 / n_total)
        mean = st_scr[0:1, :] * inv_n
        msq = st_scr[1:2, :] * inv_n
        var = msq - mean * mean
        scale = g_ref[...] * jax.lax.rsqrt(var + 1e-5)
        shift = b_ref[...] - mean * scale
        o_ref[...] = f_scr[pl.ds(j * AROWS, AROWS), :] * scale + shift


def kernel(x, vertices, conv_w, conv_b, r_w1, r_b1, r_w2, r_b2, r_w3, r_b3,
           lin_w, lin_b, bn_gamma, bn_beta):
    n = x.shape[0]
    xf = x.reshape(n, IMG).astype(jnp.bfloat16)

    a_mat, bias, wcl = pl.pallas_call(
        _prep_body,
        out_shape=[
            jax.ShapeDtypeStruct((VIEWS, VIEWS), jnp.float32),
            jax.ShapeDtypeStruct((VIEWS, FEAT), jnp.float32),
            jax.ShapeDtypeStruct((IMG, FEAT), jnp.float32),
        ],
    )(vertices, conv_w.reshape(1, 27), conv_b.reshape(1, 1),
      r_w1.T, r_b1.reshape(1, 10), r_w2.T, r_b2.reshape(1, 10),
      r_w3.T, r_b3.reshape(1, 1), lin_w, lin_b.reshape(1, FEAT))

    wcl = wcl.astype(jnp.bfloat16)
    nsteps = n // ROWS
    napply = n // AROWS
    out = pl.pallas_call(
        functools.partial(_mega_body, nsteps, n),
        grid=(nsteps + napply,),
        in_specs=[
            pl.BlockSpec((ROWS, IMG), lambda i: (jnp.minimum(i, nsteps - 1), 0)),
            pl.BlockSpec((IMG, FEAT), lambda i: (0, 0)),
            pl.BlockSpec((VIEWS, VIEWS), lambda i: (0, 0)),
            pl.BlockSpec((VIEWS, FEAT), lambda i: (0, 0)),
            pl.BlockSpec((1, FEAT), lambda i: (0, 0)),
            pl.BlockSpec((1, FEAT), lambda i: (0, 0)),
        ],
        out_specs=pl.BlockSpec(
            (AROWS, FEAT), lambda i: (jnp.maximum(i - nsteps, 0), 0)),
        out_shape=jax.ShapeDtypeStruct((n, FEAT), jnp.float32),
        scratch_shapes=[
            pltpu.VMEM((n, FEAT), jnp.float32),
            pltpu.VMEM((2, FEAT), jnp.float32),
        ],
        compiler_params=pltpu.CompilerParams(
            dimension_semantics=("arbitrary",)),
    )(xf, wcl, a_mat, bias,
      bn_gamma.reshape(1, FEAT), bn_beta.reshape(1, FEAT))

    return out.reshape(n, 1, FEAT)
